# baseline (speedup 1.0000x reference)
import functools

import jax
import jax.numpy as jnp
from jax import lax
from jax.experimental import pallas as pl
from jax.experimental.pallas import tpu as pltpu
from jax.experimental.pallas import tpu_sc as plsc

NUM_FIELDS = 26
EMBED = 16
BATCH = 16384
FIELD_SIZE = 100000
TOTAL = FIELD_SIZE * NUM_FIELDS
FC_PAD = 2600960
NUM_WORKERS = 32
PER_W = BATCH // NUM_WORKERS
CHUNK = 128
NCHUNK = PER_W // CHUNK
BLK = 16
NBLK = CHUNK // BLK
NIDX = NUM_FIELDS * CHUNK


@functools.partial(
    pl.kernel,
    mesh=plsc.VectorSubcoreMesh(core_axis_name="c", subcore_axis_name="s"),
    out_type=jax.ShapeDtypeStruct((BATCH,), jnp.float32),
    compiler_params=pltpu.CompilerParams(
        needs_layout_passes=False, use_tc_tiling_on_sc=False),
    scratch_types=[
        pltpu.VMEM((NUM_FIELDS, CHUNK), jnp.int32),
        pltpu.VMEM((2, NIDX), jnp.int32),
        pltpu.VMEM((2, NIDX, EMBED), jnp.float32),
        pltpu.VMEM((2, NIDX), jnp.float32),
        pltpu.VMEM((16,), jnp.float32),
        pltpu.VMEM((BLK, EMBED), jnp.float32),
        pltpu.VMEM((CHUNK,), jnp.float32),
        pltpu.SemaphoreType.DMA,
        pltpu.SemaphoreType.DMA,
    ],
)
def _fm_kernel(xT_hbm, emb_hbm, fc_hbm, bias_hbm, out_hbm,
               xT_v, idx_v, rows_v, fcv_v, bias_v, tmp_v, out_v, sem0, sem1):
    cid = lax.axis_index("c")
    sid = lax.axis_index("s")
    wid = sid * 2 + cid
    base = wid * PER_W
    sems = (sem0, sem1)

    pltpu.sync_copy(bias_hbm, bias_v)

    def stage(c):
        buf = c % 2
        cbase = base + c * CHUNK
        pltpu.sync_copy(xT_hbm.at[:, pl.ds(cbase, CHUNK)], xT_v)

        def idx_body(f, _):
            off = f * FIELD_SIZE

            def p_body(p, _):
                idx_v[buf, pl.ds(f * CHUNK + p * 16, 16)] = (
                    xT_v[f, pl.ds(p * 16, 16)] + off)
                return 0

            lax.fori_loop(0, CHUNK // 16, p_body, 0, unroll=True)
            return 0

        lax.fori_loop(0, NUM_FIELDS, idx_body, 0)

        return [
            pltpu.async_copy(emb_hbm.at[idx_v.at[buf]],
                             rows_v.at[buf], sems[buf]),
            pltpu.async_copy(fc_hbm.at[idx_v.at[buf]],
                             fcv_v.at[buf], sems[buf]),
        ]

    lanes = lax.iota(jnp.int32, 16)

    def compute(c, descs):
        buf = c % 2
        cbase = base + c * CHUNK
        for dd in descs:
            dd.wait()

        bvec = bias_v[...]

        def blk_body(blk, _):
            eb = blk * BLK
            lacc = bvec
            for f in range(NUM_FIELDS):
                lacc = lacc + fcv_v[buf, pl.ds(f * CHUNK + eb, BLK)]

            for j in range(BLK):
                e = eb + j
                v = rows_v[buf, e, :]
                s = v
                ss = v * v
                for f in range(1, NUM_FIELDS):
                    v = rows_v[buf, f * CHUNK + e, :]
                    s = s + v
                    ss = ss + v * v
                tmp_v[j, :] = s * s - ss

            def d_body(d, acc):
                return acc + plsc.load_gather(
                    tmp_v, [lanes, jnp.full((16,), d, jnp.int32)])

            facc = lax.fori_loop(0, EMBED, d_body,
                                 jnp.zeros((16,), jnp.float32))

            z = lacc + 0.5 * facc
            out_v[pl.ds(eb, BLK)] = 1.0 / (1.0 + jnp.exp(-z))
            return 0

        lax.fori_loop(0, NBLK, blk_body, 0)

        pltpu.sync_copy(out_v, out_hbm.at[pl.ds(cbase, CHUNK)])

    descs = stage(0)
    for c in range(NCHUNK):
        nxt = stage(c + 1) if c + 1 < NCHUNK else None
        compute(c, descs)
        descs = nxt


def kernel(x, emb_table, fc_table, bias):
    xT = x.astype(jnp.int32).T
    fc = jnp.pad(fc_table, ((0, FC_PAD - TOTAL), (0, 0))).reshape(-1)
    bias16 = jnp.broadcast_to(bias.astype(jnp.float32), (16,))
    return _fm_kernel(xT, emb_table, fc, bias16)


# speedup vs baseline: 2.9621x; 2.9621x over previous
import functools

import jax
import jax.numpy as jnp
from jax import lax
from jax.experimental import pallas as pl
from jax.experimental.pallas import tpu as pltpu
from jax.experimental.pallas import tpu_sc as plsc

NUM_FIELDS = 26
EMBED = 16
BATCH = 16384
FIELD_SIZE = 100000
TOTAL = FIELD_SIZE * NUM_FIELDS
ROWS_PAD = 2600064
NTILE = ROWS_PAD // 128
GSTRIDE = NTILE * 1024
TOTAL1D = 2 * GSTRIDE
FC_PAD = 2600960
NUM_WORKERS = 32
PER_W = BATCH // NUM_WORKERS
CHUNK = 128
NCHUNK = PER_W // CHUNK
BLK = 16
NBLK = CHUNK // BLK
NIDX = NUM_FIELDS * CHUNK


@functools.partial(
    pl.kernel,
    mesh=plsc.VectorSubcoreMesh(core_axis_name="c", subcore_axis_name="s"),
    out_type=jax.ShapeDtypeStruct((BATCH,), jnp.float32),
    compiler_params=pltpu.CompilerParams(
        needs_layout_passes=False, use_tc_tiling_on_sc=False),
    scratch_types=[
        pltpu.VMEM((NUM_FIELDS, CHUNK), jnp.int32),
        pltpu.VMEM((2, NIDX), jnp.int32),
        pltpu.VMEM((2, NIDX), jnp.int32),
        pltpu.VMEM((2, EMBED, NIDX), jnp.float32),
        pltpu.VMEM((2, NIDX), jnp.float32),
        pltpu.VMEM((16,), jnp.float32),
        pltpu.VMEM((CHUNK,), jnp.float32),
        pltpu.SemaphoreType.DMA,
        pltpu.SemaphoreType.DMA,
    ],
)
def _fm_kernel(xT_hbm, emb_hbm, fc_hbm, bias_hbm, out_hbm,
               xT_v, idx_v, b_v, vals_v, fcv_v, bias_v, out_v, sem0, sem1):
    cid = lax.axis_index("c")
    sid = lax.axis_index("s")
    wid = sid * 2 + cid
    base = wid * PER_W
    sems = (sem0, sem1)

    pltpu.sync_copy(bias_hbm, bias_v)

    def stage(c):
        buf = c % 2
        cbase = base + c * CHUNK
        pltpu.sync_copy(xT_hbm.at[:, pl.ds(cbase, CHUNK)], xT_v)

        def idx_body(f, _):
            off = f * FIELD_SIZE

            def p_body(p, _):
                r = xT_v[f, pl.ds(p * 16, 16)] + off
                idx_v[buf, pl.ds(f * CHUNK + p * 16, 16)] = r
                b_v[buf, pl.ds(f * CHUNK + p * 16, 16)] = (
                    lax.shift_left(lax.shift_right_logical(r, 7), 10)
                    + (r & 127))
                return 0

            lax.fori_loop(0, CHUNK // 16, p_body, 0, unroll=True)
            return 0

        lax.fori_loop(0, NUM_FIELDS, idx_body, 0)

        descs = [pltpu.async_copy(fc_hbm.at[idx_v.at[buf]],
                                  fcv_v.at[buf], sems[buf])]
        for d in range(EMBED):
            cofs = (d // 8) * GSTRIDE + (d % 8) * 128
            descs.append(pltpu.async_copy(
                emb_hbm.at[pl.ds(cofs, TOTAL1D - cofs)].at[b_v.at[buf]],
                vals_v.at[buf].at[d], sems[buf]))
        return descs

    def compute(c, descs):
        buf = c % 2
        cbase = base + c * CHUNK
        for dd in descs:
            dd.wait()

        bvec = bias_v[...]

        def blk_body(blk, _):
            eb = blk * BLK
            lacc = bvec
            for f in range(NUM_FIELDS):
                lacc = lacc + fcv_v[buf, pl.ds(f * CHUNK + eb, BLK)]

            def d_body(d, acc):
                v = vals_v[buf, d, pl.ds(eb, BLK)]
                s = v
                ss = v * v
                for f in range(1, NUM_FIELDS):
                    v = vals_v[buf, d, pl.ds(f * CHUNK + eb, BLK)]
                    s = s + v
                    ss = ss + v * v
                return acc + (s * s - ss)

            facc = lax.fori_loop(0, EMBED, d_body,
                                 jnp.zeros((16,), jnp.float32))

            z = lacc + 0.5 * facc
            out_v[pl.ds(eb, BLK)] = 1.0 / (1.0 + jnp.exp(-z))
            return 0

        lax.fori_loop(0, NBLK, blk_body, 0)

        pltpu.sync_copy(out_v, out_hbm.at[pl.ds(cbase, CHUNK)])

    descs = stage(0)
    for c in range(NCHUNK):
        nxt = stage(c + 1) if c + 1 < NCHUNK else None
        compute(c, descs)
        descs = nxt


def kernel(x, emb_table, fc_table, bias):
    xT = x.astype(jnp.int32).T
    emb1d = (jnp.pad(emb_table, ((0, ROWS_PAD - TOTAL), (0, 0)))
             .reshape(NTILE, 128, 2, 8)
             .transpose(2, 0, 3, 1)
             .reshape(-1))
    fc = jnp.pad(fc_table, ((0, FC_PAD - TOTAL), (0, 0))).reshape(-1)
    bias16 = jnp.broadcast_to(bias.astype(jnp.float32), (16,))
    return _fm_kernel(xT, emb1d, fc, bias16)
